# baseline (speedup 1.0000x reference)
import jax
import jax.numpy as jnp
import numpy as np
from jax.experimental import pallas as pl

B = 16
H = 224
W = 224
PATCH = 14
PH = H // PATCH
PW = W // PATCH
P = PH * PW
D = 1024
L = 2
NF = 4
NG = 4
NN = B // NG
NRN = NN - 1
NSL = NN // 2
NRT = NRN + 2
KSEL = 5


def _pool_matrix() -> np.ndarray:
    a = np.zeros((PH, PH), np.float32)
    for i in range(PH):
        lo, hi = max(0, i - 1), min(PH - 1, i + 1)
        a[i, lo:hi + 1] = 1.0 / (hi - lo + 1)
    return np.kron(a, a).astype(np.float32)


def _resize_matrix() -> np.ndarray:
    scale = H / PH
    out = np.zeros((H, PH), np.float32)
    for i in range(H):
        x = (i + 0.5) / scale - 0.5
        w = np.maximum(0.0, 1.0 - np.abs(x - np.arange(PH)))
        out[i] = w / w.sum()
    return out


_POOL = _pool_matrix()
_RESIZE = _resize_matrix()
_A1 = _RESIZE[:, np.arange(P) // PW].astype(np.float32)
_A2 = _RESIZE[:, np.arange(P) % PW].astype(np.float32).T.copy()


def _bf16_dot(x, y, dims):
    return jax.lax.dot_general(
        x.astype(jnp.bfloat16), y.astype(jnp.bfloat16), dims,
        preferred_element_type=jnp.float32)


def _k1_body(patches_ref, wp_ref, bp_ref, wl_ref, bl_ref, pool_ref,
             feats_ref, sqb_ref, sql_ref):
    t = _bf16_dot(patches_ref[...].reshape(NG * P, 3 * PATCH * PATCH),
                  wp_ref[...], (((1,), (0,)), ((), ()))) + bp_ref[...]
    pool = pool_ref[...]
    ones = jnp.ones((D, 128), jnp.bfloat16)

    def emit(i, f):
        feats_ref[i] = f.reshape(NG, P, D).astype(jnp.bfloat16)
        f2 = f * f
        hi = f2.astype(jnp.bfloat16)
        lo = (f2 - hi.astype(jnp.float32)).astype(jnp.bfloat16)
        dims = (((1,), (0,)), ((), ()))
        sq = (jax.lax.dot_general(hi, ones, dims,
                                  preferred_element_type=jnp.float32)
              + jax.lax.dot_general(lo, ones, dims,
                                    preferred_element_type=jnp.float32))
        sqb_ref[i] = sq.reshape(NG, P, 128)
        sql_ref[i, 0] = jnp.sum(f2, axis=1)

    for l in range(L):
        x = _bf16_dot(t, wl_ref[l], (((1,), (0,)), ((), ()))) + bl_ref[l]
        f = jax.nn.gelu(x)
        emit(l, f)
        pooled = jnp.concatenate(
            [_bf16_dot(pool, f[g * P:(g + 1) * P], (((1,), (0,)), ((), ())))
             for g in range(NG)], axis=0)
        emit(2 + l, pooled)


def _node_pair(rnd, s):
    diag_n = (rnd - NRN) * NSL + s
    na = jnp.where(rnd >= NRN, diag_n,
                   jnp.where(s == 0, NN - 1, (rnd + s) % NRN))
    nb = jnp.where(rnd >= NRN, diag_n, (rnd - s) % NRN)
    return na, nb


def _k2_body(feats_ref, sqb_all_ref, ma_ref, mb_ref):
    rnd = pl.program_id(1)
    s = pl.program_id(2)
    na, nb = _node_pair(rnd, s)
    qa = feats_ref[0, pl.ds(na * NG, NG)].reshape(NG * P, D)
    rb = feats_ref[0, pl.ds(nb * NG, NG)].reshape(NG * P, D)
    dims = (((1,), (1,)), ((), ()))
    sqa = sqb_all_ref[0, pl.ds(na * NG, NG)].reshape(NG * P, 128)
    sqb = sqb_all_ref[0, pl.ds(nb * NG, NG)].reshape(NG * P, 128)
    ncopy = NG * P // 128
    st = jax.lax.dot_general(rb, qa, dims, preferred_element_type=jnp.float32)
    sqbw = jnp.concatenate([sqb] * ncopy, axis=1)
    ma_ref[0, 0] = jnp.min((sqbw - 2.0 * st).reshape(NG, P, NG * P), axis=1)

    @pl.when(rnd < NRN)
    def _cross():
        s2 = jax.lax.dot_general(qa, rb, dims,
                                 preferred_element_type=jnp.float32)
        sqaw = jnp.concatenate([sqa] * ncopy, axis=1)
        mb_ref[0, 0] = jnp.min((sqaw - 2.0 * s2).reshape(NG, P, NG * P), axis=1)


def _k3a_body(ma_ref, mb_ref, sql_ref, a1_ref, a2_ref, pix_ref, final_ref):
    ma = ma_ref[...]
    mb = mb_ref[...]
    sql = sql_ref[...][:, None]
    big = jnp.float32(3.0e38)
    img = jax.lax.broadcasted_iota(jnp.int32, ma.shape, 3) // P
    rnd = jax.lax.broadcasted_iota(jnp.int32, ma.shape, 1)
    opp = jax.lax.broadcasted_iota(jnp.int32, ma.shape, 2)
    node = img // NG
    side = (node - rnd) % NRN
    is_a = (node == NN - 1) | ((side >= 1) & (side <= NSL - 1))
    cross = jnp.where(is_a, ma, mb)
    diag_ok = (rnd == NRN + node // NSL) & (opp != img % NG)
    vals = jnp.where(rnd < NRN, cross, jnp.where(diag_ok, ma, big))
    cidx = rnd * NG + opp
    total = jnp.zeros((NF, 1, 1, B * P), jnp.float32)
    for _ in range(KSEL):
        mv = jnp.min(vals, axis=(1, 2), keepdims=True)
        total = total + jnp.sqrt(jnp.maximum(mv + sql, 1e-12))
        eq = vals <= mv
        idx = jnp.where(eq, cidx, NG * NRT)
        first = cidx == jnp.min(idx, axis=(1, 2), keepdims=True)
        vals = jnp.where(first, big, vals)
    sc = jnp.mean(total, axis=0)[0, 0] * (1.0 / KSEL)
    a1 = a1_ref[...]
    a2 = a2_ref[...]
    final_ref[0] = jnp.stack([jnp.max(sc[b * P:(b + 1) * P]) for b in range(B)])
    for b in range(B):
        x = (a1 * sc[b * P:(b + 1) * P][None, :]).astype(jnp.bfloat16)
        pix_ref[b] = jax.lax.dot_general(
            x, a2.astype(jnp.bfloat16), (((1,), (0,)), ((), ())),
            preferred_element_type=jnp.float32)


@jax.jit
def kernel(pixel_values, W_patch, b_patch, W_layers, b_layers):
    patches = pixel_values.astype(jnp.bfloat16).reshape(B, 3, PH, PATCH, PW, PATCH)
    patches = patches.transpose(0, 2, 4, 1, 3, 5).reshape(B, P, 3 * PATCH * PATCH)
    cdim = patches.shape[-1]

    feats, sqb, sql = pl.pallas_call(
        _k1_body,
        grid=(B // NG,),
        in_specs=[
            pl.BlockSpec((NG, P, cdim), lambda b: (b, 0, 0)),
            pl.BlockSpec((cdim, D), lambda b: (0, 0)),
            pl.BlockSpec((D,), lambda b: (0,)),
            pl.BlockSpec((L, D, D), lambda b: (0, 0, 0)),
            pl.BlockSpec((L, D), lambda b: (0, 0)),
            pl.BlockSpec((P, P), lambda b: (0, 0)),
        ],
        out_specs=[
            pl.BlockSpec((NF, NG, P, D), lambda b: (0, b, 0, 0)),
            pl.BlockSpec((NF, NG, P, 128), lambda b: (0, b, 0, 0)),
            pl.BlockSpec((NF, 1, NG * P), lambda b: (0, 0, b)),
        ],
        out_shape=[
            jax.ShapeDtypeStruct((NF, B, P, D), jnp.bfloat16),
            jax.ShapeDtypeStruct((NF, B, P, 128), jnp.float32),
            jax.ShapeDtypeStruct((NF, 1, B * P), jnp.float32),
        ],
    )(patches, W_patch, b_patch, W_layers, b_layers, jnp.asarray(_POOL))

    ma, mb = pl.pallas_call(
        _k2_body,
        grid=(NF, NRT, NSL),
        in_specs=[
            pl.BlockSpec((1, B, P, D), lambda f, r, s: (f, 0, 0, 0)),
            pl.BlockSpec((1, B, P, 128), lambda f, r, s: (f, 0, 0, 0)),
        ],
        out_specs=[
            pl.BlockSpec((1, 1, NG, NG * P),
                         lambda f, r, s: (f, r, 0, _node_pair(r, s)[0])),
            pl.BlockSpec((1, 1, NG, NG * P),
                         lambda f, r, s: (f, r, 0, _node_pair(r, s)[1])),
        ],
        out_shape=[
            jax.ShapeDtypeStruct((NF, NRT, NG, B * P), jnp.float32),
            jax.ShapeDtypeStruct((NF, NRT, NG, B * P), jnp.float32),
        ],
    )(feats, sqb)

    pix, final = pl.pallas_call(
        _k3a_body,
        in_specs=[
            pl.BlockSpec((NF, NRT, NG, B * P), lambda: (0, 0, 0, 0)),
            pl.BlockSpec((NF, NRT, NG, B * P), lambda: (0, 0, 0, 0)),
            pl.BlockSpec((NF, 1, B * P), lambda: (0, 0, 0)),
            pl.BlockSpec((H, P), lambda: (0, 0)),
            pl.BlockSpec((P, H), lambda: (0, 0)),
        ],
        out_specs=[
            pl.BlockSpec((B, H, W), lambda: (0, 0, 0)),
            pl.BlockSpec((1, B), lambda: (0, 0)),
        ],
        out_shape=[
            jax.ShapeDtypeStruct((B, H, W), jnp.float32),
            jax.ShapeDtypeStruct((1, B), jnp.float32),
        ],
    )(ma, mb, sql, jnp.asarray(_A1), jnp.asarray(_A2))

    return final[0], pix


# speedup vs baseline: 1.0855x; 1.0855x over previous
import jax
import jax.numpy as jnp
import numpy as np
from jax.experimental import pallas as pl

B = 16
H = 224
W = 224
PATCH = 14
PH = H // PATCH
PW = W // PATCH
P = PH * PW
D = 1024
L = 2
NF = 4
NG = 4
NN = B // NG
NRN = NN - 1
NSL = NN // 2
NRT = NRN + 2
KSEL = 5


def _pool_matrix() -> np.ndarray:
    a = np.zeros((PH, PH), np.float32)
    for i in range(PH):
        lo, hi = max(0, i - 1), min(PH - 1, i + 1)
        a[i, lo:hi + 1] = 1.0 / (hi - lo + 1)
    return np.kron(a, a).astype(np.float32)


def _resize_matrix() -> np.ndarray:
    scale = H / PH
    out = np.zeros((H, PH), np.float32)
    for i in range(H):
        x = (i + 0.5) / scale - 0.5
        w = np.maximum(0.0, 1.0 - np.abs(x - np.arange(PH)))
        out[i] = w / w.sum()
    return out


_POOL = _pool_matrix()
_RESIZE = _resize_matrix()
_A1 = _RESIZE[:, np.arange(P) // PW].astype(np.float32)
_A2 = _RESIZE[:, np.arange(P) % PW].astype(np.float32).T.copy()


def _bf16_dot(x, y, dims):
    return jax.lax.dot_general(
        x.astype(jnp.bfloat16), y.astype(jnp.bfloat16), dims,
        preferred_element_type=jnp.float32)


def _k1_body(patches_ref, wp_ref, bp_ref, wl_ref, bl_ref, pool_ref,
             feats_ref, sqb_ref, sql_ref):
    t = _bf16_dot(patches_ref[...].reshape(NG * P, 3 * PATCH * PATCH),
                  wp_ref[...], (((1,), (0,)), ((), ()))) + bp_ref[...]
    pool = pool_ref[...]
    ones = jnp.ones((D, 128), jnp.bfloat16)

    def emit(i, f):
        feats_ref[i] = f.reshape(NG, P, D).astype(jnp.bfloat16)
        f2 = f * f
        hi = f2.astype(jnp.bfloat16)
        lo = (f2 - hi.astype(jnp.float32)).astype(jnp.bfloat16)
        dims = (((1,), (0,)), ((), ()))
        sq = (jax.lax.dot_general(hi, ones, dims,
                                  preferred_element_type=jnp.float32)
              + jax.lax.dot_general(lo, ones, dims,
                                    preferred_element_type=jnp.float32))
        sqb_ref[i] = sq.reshape(NG, P, 128)
        sql_ref[i, 0] = jnp.sum(f2, axis=1)

    for l in range(L):
        x = _bf16_dot(t, wl_ref[l], (((1,), (0,)), ((), ()))) + bl_ref[l]
        f = jax.nn.gelu(x)
        emit(l, f)
        pooled = jnp.concatenate(
            [_bf16_dot(pool, f[g * P:(g + 1) * P], (((1,), (0,)), ((), ())))
             for g in range(NG)], axis=0)
        emit(2 + l, pooled)


def _node_pair(rnd, s):
    diag_n = (rnd - NRN) * NSL + s
    na = jnp.where(rnd >= NRN, diag_n,
                   jnp.where(s == 0, NN - 1, (rnd + s) % NRN))
    nb = jnp.where(rnd >= NRN, diag_n, (rnd - s) % NRN)
    return na, nb


def _k2_body(feats_ref, sqb_all_ref, ma_ref, mb_ref):
    rnd = pl.program_id(1)
    s = pl.program_id(2)
    na, nb = _node_pair(rnd, s)
    qa = feats_ref[0, pl.ds(na * NG, NG)].reshape(NG * P, D)
    rb = feats_ref[0, pl.ds(nb * NG, NG)].reshape(NG * P, D)
    dims = (((1,), (1,)), ((), ()))
    sqa = sqb_all_ref[0, pl.ds(na * NG, NG)].reshape(NG * P, 128)
    sqb = sqb_all_ref[0, pl.ds(nb * NG, NG)].reshape(NG * P, 128)
    ncopy = NG * P // 128
    st = jax.lax.dot_general(rb, qa, dims, preferred_element_type=jnp.float32)
    sqbw = jnp.concatenate([sqb] * ncopy, axis=1)
    ma_ref[0, 0] = jnp.min((sqbw - 2.0 * st).reshape(NG, P, NG * P), axis=1)

    @pl.when(rnd < NRN)
    def _cross():
        s2 = jax.lax.dot_general(qa, rb, dims,
                                 preferred_element_type=jnp.float32)
        sqaw = jnp.concatenate([sqa] * ncopy, axis=1)
        mb_ref[0, 0] = jnp.min((sqaw - 2.0 * s2).reshape(NG, P, NG * P), axis=1)


def _k3a_body(ma_ref, mb_ref, sql_ref, a1_ref, a2_ref, pix_ref, final_ref):
    ma = ma_ref[...]
    mb = mb_ref[...]
    sql = sql_ref[...][:, None]
    big = jnp.float32(3.0e38)
    img = jax.lax.broadcasted_iota(jnp.int32, ma.shape, 3) // P
    rnd = jax.lax.broadcasted_iota(jnp.int32, ma.shape, 1)
    opp = jax.lax.broadcasted_iota(jnp.int32, ma.shape, 2)
    node = img // NG
    side = (node - rnd) % NRN
    is_a = (node == NN - 1) | ((side >= 1) & (side <= NSL - 1))
    cross = jnp.where(is_a, ma, mb)
    diag_ok = (rnd == NRN + node // NSL) & (opp != img % NG)
    vals = jnp.where(rnd < NRN, cross, jnp.where(diag_ok, ma, big))
    cidx = rnd * NG + opp
    total = jnp.zeros((NF, 1, 1, B * P), jnp.float32)
    for _ in range(KSEL):
        mv = jnp.min(vals, axis=(1, 2), keepdims=True)
        total = total + jnp.sqrt(jnp.maximum(mv + sql, 1e-12))
        eq = vals <= mv
        idx = jnp.where(eq, cidx, NG * NRT)
        first = cidx == jnp.min(idx, axis=(1, 2), keepdims=True)
        vals = jnp.where(first, big, vals)
    sc = jnp.mean(total, axis=0)[0, 0] * (1.0 / KSEL)
    a1 = a1_ref[...]
    a2 = a2_ref[...]
    final_ref[0] = jnp.stack([jnp.max(sc[b * P:(b + 1) * P]) for b in range(B)])
    for b in range(B):
        x = (a1 * sc[b * P:(b + 1) * P][None, :]).astype(jnp.bfloat16)
        pix_ref[b] = jax.lax.dot_general(
            x, a2.astype(jnp.bfloat16), (((1,), (0,)), ((), ())),
            preferred_element_type=jnp.float32)


@jax.jit
def kernel(pixel_values, W_patch, b_patch, W_layers, b_layers):
    patches = pixel_values.reshape(B, 3, PH, PATCH, PW, PATCH)
    patches = patches.transpose(0, 2, 4, 1, 3, 5).reshape(B, P, 3 * PATCH * PATCH)
    cdim = patches.shape[-1]

    feats, sqb, sql = pl.pallas_call(
        _k1_body,
        grid=(B // NG,),
        in_specs=[
            pl.BlockSpec((NG, P, cdim), lambda b: (b, 0, 0)),
            pl.BlockSpec((cdim, D), lambda b: (0, 0)),
            pl.BlockSpec((D,), lambda b: (0,)),
            pl.BlockSpec((L, D, D), lambda b: (0, 0, 0)),
            pl.BlockSpec((L, D), lambda b: (0, 0)),
            pl.BlockSpec((P, P), lambda b: (0, 0)),
        ],
        out_specs=[
            pl.BlockSpec((NF, NG, P, D), lambda b: (0, b, 0, 0)),
            pl.BlockSpec((NF, NG, P, 128), lambda b: (0, b, 0, 0)),
            pl.BlockSpec((NF, 1, NG * P), lambda b: (0, 0, b)),
        ],
        out_shape=[
            jax.ShapeDtypeStruct((NF, B, P, D), jnp.bfloat16),
            jax.ShapeDtypeStruct((NF, B, P, 128), jnp.float32),
            jax.ShapeDtypeStruct((NF, 1, B * P), jnp.float32),
        ],
    )(patches, W_patch, b_patch, W_layers, b_layers, jnp.asarray(_POOL))

    ma, mb = pl.pallas_call(
        _k2_body,
        grid=(NF, NRT, NSL),
        in_specs=[
            pl.BlockSpec((1, B, P, D), lambda f, r, s: (f, 0, 0, 0)),
            pl.BlockSpec((1, B, P, 128), lambda f, r, s: (f, 0, 0, 0)),
        ],
        out_specs=[
            pl.BlockSpec((1, 1, NG, NG * P),
                         lambda f, r, s: (f, r, 0, _node_pair(r, s)[0])),
            pl.BlockSpec((1, 1, NG, NG * P),
                         lambda f, r, s: (f, r, 0, _node_pair(r, s)[1])),
        ],
        out_shape=[
            jax.ShapeDtypeStruct((NF, NRT, NG, B * P), jnp.float32),
            jax.ShapeDtypeStruct((NF, NRT, NG, B * P), jnp.float32),
        ],
    )(feats, sqb)

    pix, final = pl.pallas_call(
        _k3a_body,
        in_specs=[
            pl.BlockSpec((NF, NRT, NG, B * P), lambda: (0, 0, 0, 0)),
            pl.BlockSpec((NF, NRT, NG, B * P), lambda: (0, 0, 0, 0)),
            pl.BlockSpec((NF, 1, B * P), lambda: (0, 0, 0)),
            pl.BlockSpec((H, P), lambda: (0, 0)),
            pl.BlockSpec((P, H), lambda: (0, 0)),
        ],
        out_specs=[
            pl.BlockSpec((B, H, W), lambda: (0, 0, 0)),
            pl.BlockSpec((1, B), lambda: (0, 0)),
        ],
        out_shape=[
            jax.ShapeDtypeStruct((B, H, W), jnp.float32),
            jax.ShapeDtypeStruct((1, B), jnp.float32),
        ],
    )(ma, mb, sql, jnp.asarray(_A1), jnp.asarray(_A2))

    return final[0], pix
